# baseline (speedup 1.0000x reference)
import functools

import jax
import jax.numpy as jnp
from jax import lax
from jax.experimental import pallas as pl
from jax.experimental.pallas import tpu as pltpu
from jax.experimental.pallas import tpu_sc as plsc


@functools.lru_cache(maxsize=None)
def _make_plane_gather(vocab: int, dim: int, hist: int, batch: int,
                       num_workers: int):
    passes = dim // num_workers
    mesh = plsc.VectorSubcoreMesh(core_axis_name="c", subcore_axis_name="s")
    nc = mesh.num_cores

    @functools.partial(
        pl.kernel,
        out_type=jax.ShapeDtypeStruct((hist, dim, batch), jnp.float32),
        mesh=mesh,
        scratch_types=[
            pltpu.VMEM((vocab,), jnp.float32),
            pltpu.VMEM((2, batch), jnp.int32),
            pltpu.VMEM((2, batch), jnp.float32),
            pltpu.SemaphoreType.DMA,
            pltpu.SemaphoreType.DMA,
            pltpu.SemaphoreType.DMA,
        ],
        compiler_params=pltpu.CompilerParams(
            use_tc_tiling_on_sc=False, needs_layout_passes=False
        ),
    )
    def plane_kernel(table_t, idx_t, out_hbm, plane_v, idxc_v, gout_v,
                     isem, osem0, osem1):
        wid = lax.axis_index("s") * nc + lax.axis_index("c")
        osems = (osem0, osem1)

        def out_drain(b):
            pltpu.make_async_copy(
                gout_v.at[b], out_hbm.at[0, 0], osems[b]
            ).wait()

        for p in range(passes):
            d = wid + num_workers * p
            pltpu.sync_copy(table_t.at[d], plane_v)
            pltpu.async_copy(idx_t.at[0], idxc_v.at[0], isem)

            @pl.loop(0, hist, step=2)
            def _cols(g):
                for b in range(2):
                    l = g + b
                    pltpu.make_async_copy(
                        idx_t.at[0], idxc_v.at[b], isem
                    ).wait()

                    @pl.when(l + 1 < hist)
                    def _():
                        pltpu.async_copy(
                            idx_t.at[l + 1], idxc_v.at[1 - b], isem
                        )

                    if p == 0:
                        @pl.when(l >= 2)
                        def _():
                            out_drain(b)
                    else:
                        out_drain(b)
                    idx_col = idxc_v.at[b]
                    out_col = gout_v.at[b]

                    @pl.loop(0, batch // 16, unroll=8)
                    def _gather16(j):
                        base = j * 16
                        iv = idx_col[pl.ds(base, 16)]
                        out_col[pl.ds(base, 16)] = plsc.load_gather(
                            plane_v, [iv]
                        )

                    pltpu.async_copy(
                        gout_v.at[b], out_hbm.at[l, d], osems[b]
                    )

        for b in range(2):
            out_drain(b)

    return plane_kernel


def kernel(indices, table):
    batch, hist = indices.shape
    vocab, dim = table.shape
    info = plsc.get_sparse_core_info()
    nw = info.num_cores * info.num_subcores
    table_t = table.T
    idx_t = indices.T
    out = _make_plane_gather(vocab, dim, hist, batch, nw)(table_t, idx_t)
    return jnp.transpose(out, (2, 0, 1))


# speedup vs baseline: 1.3871x; 1.3871x over previous
import functools

import jax
import jax.numpy as jnp
from jax import lax
from jax.experimental import pallas as pl
from jax.experimental.pallas import tpu as pltpu
from jax.experimental.pallas import tpu_sc as plsc

IDX_W = 80
STREAMS = 10
B_CHUNK = 16


@functools.lru_cache(maxsize=None)
def _make_gather(num_workers: int, batch: int, hist: int, vocab: int,
                 dim: int):
    b_per_w = batch // num_workers
    rows_per_w = b_per_w * hist
    chunk_rows = B_CHUNK * hist
    n_chunks = b_per_w // B_CHUNK
    kb = rows_per_w // IDX_W
    rows_per_chunk_stream = STREAMS
    mesh = plsc.VectorSubcoreMesh(core_axis_name="c", subcore_axis_name="s")
    nc = mesh.num_cores

    @functools.partial(
        pl.kernel,
        out_type=jax.ShapeDtypeStruct((batch, hist, dim), jnp.float32),
        mesh=mesh,
        scratch_types=[
            pltpu.VMEM((kb, IDX_W), jnp.int32),
            pltpu.VMEM((2, chunk_rows, dim), jnp.float32),
            pltpu.SemaphoreType.DMA,
            pltpu.SemaphoreType.DMA,
            pltpu.SemaphoreType.DMA,
            pltpu.SemaphoreType.DMA,
        ],
        compiler_params=pltpu.CompilerParams(use_tc_tiling_on_sc=False),
    )
    def gather_kernel(table_hbm, idx_hbm, out_hbm, idx_v, rows_v,
                      gsem0, gsem1, osem0, osem1):
        wid = lax.axis_index("s") * nc + lax.axis_index("c")
        b_base = wid * b_per_w
        pltpu.sync_copy(idx_hbm.at[wid], idx_v)
        gsems = (gsem0, gsem1)
        osems = (osem0, osem1)

        def out_drain(b):
            for _ in range(B_CHUNK):
                pltpu.make_async_copy(
                    out_hbm.at[0], rows_v.at[b, pl.ds(0, hist)], osems[b]
                ).wait()

        @pl.loop(0, n_chunks, step=2)
        def _chunks(g):
            descs = []
            for b in range(2):
                c = g + b

                @pl.when(g > 0)
                def _():
                    out_drain(b)

                for s in range(STREAMS):
                    d = pltpu.async_copy(
                        table_hbm.at[idx_v.at[c * rows_per_chunk_stream + s]],
                        rows_v.at[b, pl.ds(s * IDX_W, IDX_W)],
                        gsems[b],
                    )
                    descs.append(d)
            for b in range(2):
                c = g + b
                for s in range(STREAMS):
                    descs[b * STREAMS + s].wait()
                for i in range(B_CHUNK):
                    pltpu.async_copy(
                        rows_v.at[b, pl.ds(i * hist, hist)],
                        out_hbm.at[b_base + c * B_CHUNK + i],
                        osems[b],
                    )
        for b in range(2):
            out_drain(b)

    return gather_kernel


def kernel(indices, table):
    batch, hist = indices.shape
    vocab, dim = table.shape
    info = plsc.get_sparse_core_info()
    nw = info.num_cores * info.num_subcores
    rows_per_w = (batch // nw) * hist
    idx3 = indices.reshape(nw, rows_per_w // IDX_W, IDX_W)
    return _make_gather(nw, batch, hist, vocab, dim)(table, idx3)


# speedup vs baseline: 1.5434x; 1.1127x over previous
import functools

import jax
import jax.numpy as jnp
from jax import lax
from jax.experimental import pallas as pl
from jax.experimental.pallas import tpu as pltpu
from jax.experimental.pallas import tpu_sc as plsc


@functools.lru_cache(maxsize=None)
def _make_plane_gather(vocab: int, dim: int, hist: int, batch: int,
                       num_workers: int):
    passes = dim // num_workers
    mesh = plsc.VectorSubcoreMesh(core_axis_name="c", subcore_axis_name="s")
    nc = mesh.num_cores

    @functools.partial(
        pl.kernel,
        out_type=jax.ShapeDtypeStruct((hist, dim, batch), jnp.float32),
        mesh=mesh,
        scratch_types=[
            pltpu.VMEM((vocab,), jnp.float32),
            pltpu.VMEM((2, batch), jnp.int32),
            pltpu.VMEM((2, batch), jnp.float32),
            pltpu.SemaphoreType.DMA,
            pltpu.SemaphoreType.DMA,
            pltpu.SemaphoreType.DMA,
        ],
        compiler_params=pltpu.CompilerParams(
            use_tc_tiling_on_sc=False, needs_layout_passes=False
        ),
    )
    def plane_kernel(table_t, idx_t, out_hbm, plane_v, idxc_v, gout_v,
                     isem, osem0, osem1):
        wid = lax.axis_index("s") * nc + lax.axis_index("c")
        osems = (osem0, osem1)

        def out_drain(b):
            pltpu.make_async_copy(
                gout_v.at[b], out_hbm.at[0, 0], osems[b]
            ).wait()

        for p in range(passes):
            d = wid + num_workers * p
            pltpu.sync_copy(table_t.at[d], plane_v)
            pltpu.async_copy(idx_t.at[0], idxc_v.at[0], isem)

            @pl.loop(0, hist, step=2)
            def _cols(g):
                for b in range(2):
                    l = g + b
                    pltpu.make_async_copy(
                        idx_t.at[0], idxc_v.at[b], isem
                    ).wait()

                    @pl.when(l + 1 < hist)
                    def _():
                        pltpu.async_copy(
                            idx_t.at[l + 1], idxc_v.at[1 - b], isem
                        )

                    if p == 0:
                        @pl.when(l >= 2)
                        def _():
                            out_drain(b)
                    else:
                        out_drain(b)
                    idx_col = idxc_v.at[b]
                    out_col = gout_v.at[b]

                    @plsc.parallel_loop(0, batch, step=16, unroll=8)
                    def _gather16(j):
                        iv = idx_col[pl.ds(j, 16)]
                        out_col[pl.ds(j, 16)] = plsc.load_gather(
                            plane_v, [iv]
                        )

                    pltpu.async_copy(
                        out_col, out_hbm.at[l, d], osems[b]
                    )

        for b in range(2):
            out_drain(b)

    return plane_kernel


def kernel(indices, table):
    batch, hist = indices.shape
    vocab, dim = table.shape
    info = plsc.get_sparse_core_info()
    nw = info.num_cores * info.num_subcores
    table_t = table.T
    idx_t = indices.T
    out = _make_plane_gather(vocab, dim, hist, batch, nw)(table_t, idx_t)
    return jnp.transpose(out, (2, 0, 1))
